# Initial kernel scaffold; baseline (speedup 1.0000x reference)
import jax
import jax.numpy as jnp
from jax.experimental import pallas as pl


def kernel(x, W1l, b1, W1r, g1, be1, W2l, b2, W2r, g2, be2, Wlin, blin, edge_index):
    raise NotImplementedError("write your pallas kernel here")


# speedup vs baseline: 5.4907x; 5.4907x over previous
import functools

import jax
import jax.numpy as jnp
from jax import lax
from jax.experimental import pallas as pl
from jax.experimental.pallas import tpu as pltpu
from jax.experimental.pallas import tpu_sc as plsc

_N, _E, _D, _C = 10000, 320000, 128, 47
_NP = 10240
_DH = _D // 2
_NC, _NS = 2, 16
_EW = _E // _NS
_K = 80
_CH = _EW // _K
_RT = _NP // _NS
_RB = 128
_NB = _RT // _RB


def _sc_body(with_deg, *refs):
    if with_deg:
        (x2, src3, dst3, ones_h, zrow_h, zdeg_h, p_out, deg_out,
         src_v, dst_v, rows_v, ones_v, zrow_v, zdeg_v, agg_sh, deg_sh,
         sem) = refs
    else:
        (x2, src3, dst3, zrow_h, p_out,
         src_v, dst_v, rows_v, zrow_v, agg_sh, sem) = refs

    cid = lax.axis_index("c")
    sid = lax.axis_index("s")
    base = sid * _RT

    pltpu.sync_copy(src3.at[sid], src_v)
    pltpu.sync_copy(dst3.at[sid], dst_v)
    pltpu.sync_copy(zrow_h, zrow_v)
    for k in range(_NB):
        pltpu.sync_copy(zrow_v, agg_sh.at[pl.ds(base + k * _RB, _RB)])
    if with_deg:
        pltpu.sync_copy(ones_h, ones_v)
        pltpu.sync_copy(zdeg_h, zdeg_v)

        @pl.when(cid == 0)
        def _():
            pltpu.sync_copy(zdeg_v, deg_sh.at[pl.ds(base, _RT)])

    plsc.subcore_barrier()

    def chunk(c, carry):
        pltpu.async_copy(x2.at[cid].at[src_v.at[c]], rows_v, sem).wait()
        pltpu.sync_copy(rows_v, agg_sh.at[dst_v.at[c]], add=True)
        if with_deg:
            @pl.when(cid == 0)
            def _():
                pltpu.sync_copy(ones_v, deg_sh.at[dst_v.at[c]], add=True)
        return carry

    lax.fori_loop(0, _CH, chunk, 0)
    plsc.subcore_barrier()

    for k in range(_NB):
        sl = pl.ds(base + k * _RB, _RB)
        pltpu.sync_copy(agg_sh.at[sl], zrow_v)
        pltpu.sync_copy(zrow_v, p_out.at[cid].at[sl])
    if with_deg:
        @pl.when(cid == 0)
        def _():
            pltpu.sync_copy(deg_sh.at[pl.ds(base, _RT)], zdeg_v)
            pltpu.sync_copy(zdeg_v, deg_out.at[pl.ds(base, _RT)])


def _make_sc_agg(with_deg):
    mesh = plsc.VectorSubcoreMesh(core_axis_name="c", subcore_axis_name="s",
                                  num_cores=_NC, num_subcores=_NS)
    out_type = [jax.ShapeDtypeStruct((_NC, _NP, _DH), jnp.float32)]
    scratch = [
        pltpu.VMEM((_CH, _K), jnp.int32),
        pltpu.VMEM((_CH, _K), jnp.int32),
        pltpu.VMEM((_K, _DH), jnp.float32),
    ]
    if with_deg:
        out_type.append(jax.ShapeDtypeStruct((_NP, 16), jnp.float32))
        scratch.append(pltpu.VMEM((_K, 16), jnp.float32))
    scratch.append(pltpu.VMEM((_RB, _DH), jnp.float32))
    if with_deg:
        scratch.append(pltpu.VMEM((_RT, 16), jnp.float32))
    scratch.append(pltpu.VMEM_SHARED((_NP, _DH), jnp.float32))
    if with_deg:
        scratch.append(pltpu.VMEM_SHARED((_NP, 16), jnp.float32))
    scratch.append(pltpu.SemaphoreType.DMA)
    ot = tuple(out_type) if with_deg else out_type[0]
    return pl.kernel(functools.partial(_sc_body, with_deg),
                     out_type=ot, mesh=mesh,
                     scratch_types=tuple(scratch),
                     compiler_params=pltpu.CompilerParams(
                         use_tc_tiling_on_sc=False))


def _tc1_body(p_ref, deg_ref, x_ref, wlt_ref, wrt_ref, b_ref, g_ref, be_ref,
              h_ref, hs_ref):
    deg = jnp.maximum(deg_ref[:_N], 1.0)
    inv = 1.0 / deg[:, 0:1]
    al = p_ref[0, :_N] * inv
    ar = p_ref[1, :_N] * inv
    out = (jnp.dot(al, wlt_ref[:_DH], preferred_element_type=jnp.float32)
           + jnp.dot(ar, wlt_ref[_DH:], preferred_element_type=jnp.float32)
           + jnp.dot(x_ref[...], wrt_ref[...],
                     preferred_element_type=jnp.float32)
           + b_ref[...])
    nrm2 = jnp.sum(out * out, axis=1, keepdims=True)
    out = out * lax.rsqrt(jnp.maximum(nrm2, 1e-24))
    mu = jnp.mean(out, axis=0, keepdims=True)
    var = jnp.mean((out - mu) ** 2, axis=0, keepdims=True)
    hn = (out - mu) * lax.rsqrt(var + 1e-5) * g_ref[...] + be_ref[...]
    h = jnp.maximum(hn, 0.0)
    h_ref[...] = h
    hs_ref[0] = h[:, :_DH]
    hs_ref[1] = h[:, _DH:]


def _tc2_body(q_ref, deg_ref, h_ref, wlt_ref, wrt_ref, b_ref, g_ref, be_ref,
              wlint_ref, blin_ref, o_ref):
    deg = jnp.maximum(deg_ref[:_N], 1.0)
    inv = 1.0 / deg[:, 0:1]
    al = q_ref[0, :_N] * inv
    ar = q_ref[1, :_N] * inv
    out = (jnp.dot(al, wlt_ref[:_DH], preferred_element_type=jnp.float32)
           + jnp.dot(ar, wlt_ref[_DH:], preferred_element_type=jnp.float32)
           + jnp.dot(h_ref[...], wrt_ref[...],
                     preferred_element_type=jnp.float32)
           + b_ref[...])
    nrm2 = jnp.sum(out * out, axis=1, keepdims=True)
    out = out * lax.rsqrt(jnp.maximum(nrm2, 1e-24))
    mu = jnp.mean(out, axis=0, keepdims=True)
    var = jnp.mean((out - mu) ** 2, axis=0, keepdims=True)
    hn = (out - mu) * lax.rsqrt(var + 1e-5) * g_ref[...] + be_ref[...]
    h2 = jnp.maximum(hn, 0.0) + h_ref[...]
    logits = (jnp.dot(h2, wlint_ref[...], preferred_element_type=jnp.float32)
              + blin_ref[...])
    m = jnp.max(logits, axis=1, keepdims=True)
    s = logits - m
    lse = jnp.log(jnp.sum(jnp.exp(s), axis=1, keepdims=True))
    o_ref[...] = s - lse


_tc1 = pl.pallas_call(
    _tc1_body,
    out_shape=(jax.ShapeDtypeStruct((_N, _D), jnp.float32),
               jax.ShapeDtypeStruct((_NC, _N, _DH), jnp.float32)))
_tc2 = pl.pallas_call(
    _tc2_body,
    out_shape=jax.ShapeDtypeStruct((_N, _C), jnp.float32))


def kernel(x, W1l, b1, W1r, g1, be1, W2l, b2, W2r, g2, be2, Wlin, blin,
           edge_index):
    src = edge_index[0].reshape(_NS, _CH, _K)
    dst = edge_index[1].reshape(_NS, _CH, _K)
    x2 = jnp.stack([x[:, :_DH], x[:, _DH:]])
    ones_h = jnp.ones((_K, 16), jnp.float32)
    zrow_h = jnp.zeros((_RB, _DH), jnp.float32)
    zdeg_h = jnp.zeros((_RT, 16), jnp.float32)

    sc_agg_deg = _make_sc_agg(True)
    sc_agg = _make_sc_agg(False)

    p1, degp = sc_agg_deg(x2, src, dst, ones_h, zrow_h, zdeg_h)
    h, hs = _tc1(p1, degp, x, W1l.T, W1r.T, b1[None, :], g1[None, :],
                 be1[None, :])
    p2 = sc_agg(hs, src, dst, zrow_h)
    out = _tc2(p2, degp, h, W2l.T, W2r.T, b2[None, :], g2[None, :],
               be2[None, :], Wlin.T, blin[None, :])
    return out


# speedup vs baseline: 8.9110x; 1.6229x over previous
import functools

import jax
import jax.numpy as jnp
from jax import lax
from jax.experimental import pallas as pl
from jax.experimental.pallas import tpu as pltpu
from jax.experimental.pallas import tpu_sc as plsc

_N, _E, _D, _C = 10000, 320000, 128, 47
_NP = 10240
_DH = _D // 2
_NC, _NS = 2, 16
_EW = _E // _NS
_K = 80
_CH = _EW // _K
_RT = _NP // _NS
_RB = 128
_NB = _RT // _RB


def _sc_body(with_deg, *refs):
    if with_deg:
        (x2, src3, dst3, ones_h, zrow_h, zdeg_h, p_out, deg_out,
         src_v, dst_v, rows_v, rows_w, ones_v, zrow_v, zdeg_v, agg_sh,
         deg_sh, sem, sem2) = refs
    else:
        (x2, src3, dst3, zrow_h, p_out,
         src_v, dst_v, rows_v, rows_w, zrow_v, agg_sh, sem, sem2) = refs

    cid = lax.axis_index("c")
    sid = lax.axis_index("s")
    base = sid * _RT

    pltpu.sync_copy(src3.at[sid], src_v)
    pltpu.sync_copy(dst3.at[sid], dst_v)
    pltpu.sync_copy(zrow_h, zrow_v)
    for k in range(_NB):
        pltpu.sync_copy(zrow_v, agg_sh.at[pl.ds(base + k * _RB, _RB)])
    if with_deg:
        pltpu.sync_copy(ones_h, ones_v)
        pltpu.sync_copy(zdeg_h, zdeg_v)
        pltpu.sync_copy(zdeg_v, deg_sh.at[pl.ds(base, _RT)])
    plsc.subcore_barrier()

    rows = (rows_v, rows_w)
    sems = (sem, sem2)
    pltpu.async_copy(x2.at[cid].at[src_v.at[0]], rows[0], sems[0])

    def chunk(c, carry):
        for b in range(2):
            cc = c + b
            nxt = cc + 1

            @pl.when(nxt < _CH)
            def _():
                pltpu.async_copy(x2.at[cid].at[src_v.at[nxt]],
                                 rows[1 - b], sems[1 - b])

            pltpu.make_async_copy(x2.at[cid].at[src_v.at[cc]],
                                  rows[b], sems[b]).wait()
            pltpu.sync_copy(rows[b], agg_sh.at[dst_v.at[cc]], add=True)
            if with_deg:
                mine = lax.select(cid == 0, cc < _CH // 2, cc >= _CH // 2)

                @pl.when(mine)
                def _():
                    pltpu.sync_copy(ones_v, deg_sh.at[dst_v.at[cc]],
                                    add=True)
        return carry

    lax.fori_loop(0, _CH // 2, lambda i, c: chunk(2 * i, c), 0)
    plsc.subcore_barrier()

    for k in range(_NB):
        sl = pl.ds(base + k * _RB, _RB)
        pltpu.sync_copy(agg_sh.at[sl], zrow_v)
        pltpu.sync_copy(zrow_v, p_out.at[cid].at[sl])
    if with_deg:
        pltpu.sync_copy(deg_sh.at[pl.ds(base, _RT)], zdeg_v)
        pltpu.sync_copy(zdeg_v, deg_out.at[cid].at[pl.ds(base, _RT)])


def _make_sc_agg(with_deg):
    mesh = plsc.VectorSubcoreMesh(core_axis_name="c", subcore_axis_name="s",
                                  num_cores=_NC, num_subcores=_NS)
    out_type = [jax.ShapeDtypeStruct((_NC, _NP, _DH), jnp.float32)]
    scratch = [
        pltpu.VMEM((_CH, _K), jnp.int32),
        pltpu.VMEM((_CH, _K), jnp.int32),
        pltpu.VMEM((_K, _DH), jnp.float32),
        pltpu.VMEM((_K, _DH), jnp.float32),
    ]
    if with_deg:
        out_type.append(jax.ShapeDtypeStruct((_NC, _NP, 16), jnp.float32))
        scratch.append(pltpu.VMEM((_K, 16), jnp.float32))
    scratch.append(pltpu.VMEM((_RB, _DH), jnp.float32))
    if with_deg:
        scratch.append(pltpu.VMEM((_RT, 16), jnp.float32))
    scratch.append(pltpu.VMEM_SHARED((_NP, _DH), jnp.float32))
    if with_deg:
        scratch.append(pltpu.VMEM_SHARED((_NP, 16), jnp.float32))
    scratch.append(pltpu.SemaphoreType.DMA)
    scratch.append(pltpu.SemaphoreType.DMA)
    ot = tuple(out_type) if with_deg else out_type[0]
    return pl.kernel(functools.partial(_sc_body, with_deg),
                     out_type=ot, mesh=mesh,
                     scratch_types=tuple(scratch),
                     compiler_params=pltpu.CompilerParams(
                         use_tc_tiling_on_sc=False))


def _tc1_body(p_ref, deg_ref, x_ref, wlt_ref, wrt_ref, b_ref, g_ref, be_ref,
              h_ref, hs_ref):
    deg = jnp.maximum(deg_ref[0, :_N] + deg_ref[1, :_N], 1.0)
    inv = 1.0 / deg[:, 0:1]
    al = p_ref[0, :_N] * inv
    ar = p_ref[1, :_N] * inv
    out = (jnp.dot(al, wlt_ref[:_DH], preferred_element_type=jnp.float32)
           + jnp.dot(ar, wlt_ref[_DH:], preferred_element_type=jnp.float32)
           + jnp.dot(x_ref[...], wrt_ref[...],
                     preferred_element_type=jnp.float32)
           + b_ref[...])
    nrm2 = jnp.sum(out * out, axis=1, keepdims=True)
    out = out * lax.rsqrt(jnp.maximum(nrm2, 1e-24))
    mu = jnp.mean(out, axis=0, keepdims=True)
    var = jnp.mean((out - mu) ** 2, axis=0, keepdims=True)
    hn = (out - mu) * lax.rsqrt(var + 1e-5) * g_ref[...] + be_ref[...]
    h = jnp.maximum(hn, 0.0)
    h_ref[...] = h
    hs_ref[0] = h[:, :_DH]
    hs_ref[1] = h[:, _DH:]


def _tc2_body(q_ref, deg_ref, h_ref, wlt_ref, wrt_ref, b_ref, g_ref, be_ref,
              wlint_ref, blin_ref, o_ref):
    deg = jnp.maximum(deg_ref[0, :_N] + deg_ref[1, :_N], 1.0)
    inv = 1.0 / deg[:, 0:1]
    al = q_ref[0, :_N] * inv
    ar = q_ref[1, :_N] * inv
    out = (jnp.dot(al, wlt_ref[:_DH], preferred_element_type=jnp.float32)
           + jnp.dot(ar, wlt_ref[_DH:], preferred_element_type=jnp.float32)
           + jnp.dot(h_ref[...], wrt_ref[...],
                     preferred_element_type=jnp.float32)
           + b_ref[...])
    nrm2 = jnp.sum(out * out, axis=1, keepdims=True)
    out = out * lax.rsqrt(jnp.maximum(nrm2, 1e-24))
    mu = jnp.mean(out, axis=0, keepdims=True)
    var = jnp.mean((out - mu) ** 2, axis=0, keepdims=True)
    hn = (out - mu) * lax.rsqrt(var + 1e-5) * g_ref[...] + be_ref[...]
    h2 = jnp.maximum(hn, 0.0) + h_ref[...]
    logits = (jnp.dot(h2, wlint_ref[...], preferred_element_type=jnp.float32)
              + blin_ref[...])
    m = jnp.max(logits, axis=1, keepdims=True)
    s = logits - m
    lse = jnp.log(jnp.sum(jnp.exp(s), axis=1, keepdims=True))
    o_ref[...] = s - lse


_tc1 = pl.pallas_call(
    _tc1_body,
    out_shape=(jax.ShapeDtypeStruct((_N, _D), jnp.float32),
               jax.ShapeDtypeStruct((_NC, _N, _DH), jnp.float32)))
_tc2 = pl.pallas_call(
    _tc2_body,
    out_shape=jax.ShapeDtypeStruct((_N, _C), jnp.float32))


def kernel(x, W1l, b1, W1r, g1, be1, W2l, b2, W2r, g2, be2, Wlin, blin,
           edge_index):
    src = edge_index[0].reshape(_NS, _CH, _K)
    dst = edge_index[1].reshape(_NS, _CH, _K)
    x2 = jnp.stack([x[:, :_DH], x[:, _DH:]])
    ones_h = jnp.ones((_K, 16), jnp.float32)
    zrow_h = jnp.zeros((_RB, _DH), jnp.float32)
    zdeg_h = jnp.zeros((_RT, 16), jnp.float32)

    sc_agg_deg = _make_sc_agg(True)
    sc_agg = _make_sc_agg(False)

    p1, degp = sc_agg_deg(x2, src, dst, ones_h, zrow_h, zdeg_h)
    h, hs = _tc1(p1, degp, x, W1l.T, W1r.T, b1[None, :], g1[None, :],
                 be1[None, :])
    p2 = sc_agg(hs, src, dst, zrow_h)
    out = _tc2(p2, degp, h, W2l.T, W2r.T, b2[None, :], g2[None, :],
               be2[None, :], Wlin.T, blin[None, :])
    return out
